# Initial kernel scaffold; baseline (speedup 1.0000x reference)
import jax
import jax.numpy as jnp
from jax.experimental import pallas as pl


def kernel(x, x_deepwalk, edge_index, W1, b1, W2, b2, Wd1, bd1, Wd2, bd2):
    raise NotImplementedError("write your pallas kernel here")


# speedup vs baseline: 19.5733x; 19.5733x over previous
import functools

import jax
import jax.numpy as jnp
from jax import lax
from jax.experimental import pallas as pl
from jax.experimental.pallas import tpu as pltpu
from jax.experimental.pallas import tpu_sc as plsc

NC = 2
NS = 16
NW = NC * NS
CHUNK = 128
BN = 1000


def _mesh():
    return plsc.VectorSubcoreMesh(core_axis_name="c", subcore_axis_name="s")


_SC_PARAMS = pltpu.CompilerParams(use_tc_tiling_on_sc=False)


STG = 200
ZB = 256


def _make_deg(n, e_pad):
    ch = e_pad // (NW * CHUNK)
    nco = n // BN

    @functools.partial(
        pl.kernel,
        out_type=jax.ShapeDtypeStruct((NC * n,), jnp.float32),
        mesh=_mesh(),
        compiler_params=_SC_PARAMS,
        scratch_types=[
            pltpu.VMEM((CHUNK,), jnp.int32),
            pltpu.VMEM((CHUNK,), jnp.float32),
            pltpu.VMEM((ZB,), jnp.float32),
            pltpu.VMEM_SHARED((n + 8,), jnp.float32),
        ],
    )
    def deg_k(dst_hbm, out_hbm, didx, ones_v, zbuf, acc):
        cid = lax.axis_index("c")
        sid = lax.axis_index("s")
        for j in range(CHUNK // 16):
            ones_v[pl.ds(j * 16, 16)] = jnp.ones((16,), jnp.float32)
        for j in range(ZB // 16):
            zbuf[pl.ds(j * 16, 16)] = jnp.zeros((16,), jnp.float32)

        @pl.when(sid < nco)
        def _():
            for k in range(BN // STG):
                pltpu.sync_copy(zbuf.at[pl.ds(0, STG)],
                                acc.at[pl.ds(sid * BN + k * STG, STG)])

        @pl.when(sid == nco)
        def _():
            pltpu.sync_copy(zbuf.at[pl.ds(0, 8)], acc.at[pl.ds(n, 8)])

        plsc.subcore_barrier()

        e0 = (cid * NS + sid) * ch * CHUNK

        def body(i, carry):
            pltpu.sync_copy(dst_hbm.at[pl.ds(e0 + i * CHUNK, CHUNK)], didx)
            pltpu.sync_copy(ones_v, acc.at[didx], add=True)
            return carry

        lax.fori_loop(0, ch, body, 0)
        plsc.subcore_barrier()

        @pl.when(sid < nco)
        def _():
            for k in range(BN // STG):
                r = sid * BN + k * STG
                pltpu.sync_copy(acc.at[pl.ds(r, STG)], zbuf.at[pl.ds(0, STG)])
                pltpu.sync_copy(zbuf.at[pl.ds(0, STG)],
                                out_hbm.at[pl.ds(cid * n + r, STG)])

    return deg_k


def _make_prop(n, e_pad, w):
    ch = e_pad // (NW * CHUNK)
    nco = n // BN

    @functools.partial(
        pl.kernel,
        out_type=jax.ShapeDtypeStruct((NC * n, w), jnp.float32),
        mesh=_mesh(),
        compiler_params=_SC_PARAMS,
        scratch_types=[
            pltpu.VMEM((CHUNK,), jnp.int32),
            pltpu.VMEM((CHUNK,), jnp.int32),
            pltpu.VMEM((CHUNK, w), jnp.float32),
            pltpu.VMEM((ZB, w), jnp.float32),
            pltpu.VMEM_SHARED((n + 8, w), jnp.float32),
            pltpu.SemaphoreType.DMA,
        ],
    )
    def prop_k(tab_hbm, src_hbm, dst_hbm, out_hbm,
               sidx, didx, rows, zbuf, acc, sem):
        cid = lax.axis_index("c")
        sid = lax.axis_index("s")

        def zfill(i, carry):
            for j in range(w // 16):
                zbuf[i, pl.ds(j * 16, 16)] = jnp.zeros((16,), jnp.float32)
            return carry

        lax.fori_loop(0, ZB, zfill, 0)

        @pl.when(sid < nco)
        def _():
            for k in range(BN // STG):
                pltpu.sync_copy(zbuf.at[pl.ds(0, STG), :],
                                acc.at[pl.ds(sid * BN + k * STG, STG), :])

        @pl.when(sid == nco)
        def _():
            pltpu.sync_copy(zbuf.at[pl.ds(0, 8), :], acc.at[pl.ds(n, 8), :])

        plsc.subcore_barrier()

        e0 = (cid * NS + sid) * ch * CHUNK

        def body(i, carry):
            off = e0 + i * CHUNK
            pltpu.sync_copy(src_hbm.at[pl.ds(off, CHUNK)], sidx)
            pltpu.sync_copy(dst_hbm.at[pl.ds(off, CHUNK)], didx)
            pltpu.async_copy(tab_hbm.at[sidx], rows, sem).wait()
            pltpu.sync_copy(rows, acc.at[didx], add=True)
            return carry

        lax.fori_loop(0, ch, body, 0)
        plsc.subcore_barrier()

        @pl.when(sid < nco)
        def _():
            for k in range(BN // STG):
                r = sid * BN + k * STG
                pltpu.sync_copy(acc.at[pl.ds(r, STG), :],
                                zbuf.at[pl.ds(0, STG), :])
                pltpu.sync_copy(zbuf.at[pl.ds(0, STG), :],
                                out_hbm.at[pl.ds(cid * n + r, STG), :])

    return prop_k


def _mm_body(x_ref, xd_ref, degT_ref, W1_ref, Wd1_ref, up_ref):
    h = W1_ref.shape[1]
    hd = Wd1_ref.shape[1]
    dinv = lax.rsqrt(1.0 + degT_ref[:, 0:1] + degT_ref[:, 1:2])
    a = jnp.dot(x_ref[...], W1_ref[...], preferred_element_type=jnp.float32)
    b = jnp.dot(xd_ref[...], Wd1_ref[...], preferred_element_type=jnp.float32)
    up_ref[:, 0:h] = a * dinv
    up_ref[:, h:h + hd] = b * dinv


def _mid_body(agg0_ref, agg1_ref, up_ref, degT_ref, b1_ref, Wc_ref, z_ref):
    dinv = lax.rsqrt(1.0 + degT_ref[:, 0:1] + degT_ref[:, 1:2])
    v = jnp.maximum(
        dinv * (agg0_ref[...] + agg1_ref[...] + up_ref[...]) + b1_ref[...], 0.0)
    z_ref[...] = dinv * jnp.dot(v, Wc_ref[...],
                                preferred_element_type=jnp.float32)


def _out_body(agg0_ref, agg1_ref, z_ref, degT_ref, bc_ref, o_ref):
    c = o_ref.shape[1]
    dinv = lax.rsqrt(1.0 + degT_ref[:, 0:1] + degT_ref[:, 1:2])
    pre = dinv * (agg0_ref[...] + agg1_ref[...] + z_ref[...]) + bc_ref[...]
    col = lax.broadcasted_iota(jnp.int32, pre.shape, 1)
    prem = jnp.where(col < c, pre, -1e30)
    m = jnp.max(prem, axis=1, keepdims=True)
    ex = jnp.where(col < c, jnp.exp(prem - m), 0.0)
    lse = jnp.log(jnp.sum(ex, axis=1, keepdims=True)) + m
    o_ref[...] = (pre - lse)[:, 0:c]


def _row_spec(w):
    return pl.BlockSpec((BN, w), lambda i: (i, 0))


def _full_spec(a, b):
    return pl.BlockSpec((a, b), lambda i: (0, 0))


def kernel(x, x_deepwalk, edge_index, W1, b1, W2, b2, Wd1, bd1, Wd2, bd2):
    n, d = x.shape
    dw = x_deepwalk.shape[1]
    e = edge_index.shape[1]
    h = W1.shape[1]
    hd = Wd1.shape[1]
    c = W2.shape[1]
    wu = h + hd
    wz = 16
    f32 = jnp.float32

    ch = -(-e // (NW * CHUNK))
    e_pad = ch * NW * CHUNK
    fill = jnp.full((e_pad - e,), n, jnp.int32)
    srcp = jnp.concatenate([edge_index[0], fill])
    dstp = jnp.concatenate([edge_index[1], fill])

    b1cat = jnp.concatenate([b1, bd1]).reshape(1, wu)
    wc = jnp.concatenate([0.2 * W2, 0.1 * Wd2], axis=0)
    wc16 = jnp.pad(wc, ((0, 0), (0, wz - c)))
    bc16 = jnp.pad(0.2 * b2 + 0.1 * bd2, (0, wz - c)).reshape(1, wz)

    deg2 = _make_deg(n, e_pad)(dstp)
    degT = jnp.stack([deg2[:n], deg2[n:]], axis=1)

    grid = n // BN
    up = pl.pallas_call(
        _mm_body,
        grid=(grid,),
        in_specs=[_row_spec(d), _row_spec(dw), _row_spec(2),
                  _full_spec(d, h), _full_spec(dw, hd)],
        out_specs=_row_spec(wu),
        out_shape=jax.ShapeDtypeStruct((n, wu), f32),
    )(x, x_deepwalk, degT, W1, Wd1)

    up_pad = jnp.concatenate([up, jnp.zeros((8, wu), f32)])
    agg = _make_prop(n, e_pad, wu)(up_pad, srcp, dstp)

    z16 = pl.pallas_call(
        _mid_body,
        grid=(grid,),
        in_specs=[_row_spec(wu), _row_spec(wu), _row_spec(wu), _row_spec(2),
                  _full_spec(1, wu), _full_spec(wu, wz)],
        out_specs=_row_spec(wz),
        out_shape=jax.ShapeDtypeStruct((n, wz), f32),
    )(agg[:n], agg[n:], up, degT, b1cat, wc16)

    z_pad = jnp.concatenate([z16, jnp.zeros((8, wz), f32)])
    agg2 = _make_prop(n, e_pad, wz)(z_pad, srcp, dstp)

    out = pl.pallas_call(
        _out_body,
        grid=(grid,),
        in_specs=[_row_spec(wz), _row_spec(wz), _row_spec(wz), _row_spec(2),
                  _full_spec(1, wz)],
        out_specs=_row_spec(c),
        out_shape=jax.ShapeDtypeStruct((n, c), f32),
    )(agg2[:n], agg2[n:], z16, degT, bc16)
    return out


# speedup vs baseline: 20.5311x; 1.0489x over previous
import functools

import jax
import jax.numpy as jnp
from jax import lax
from jax.experimental import pallas as pl
from jax.experimental.pallas import tpu as pltpu
from jax.experimental.pallas import tpu_sc as plsc

NC = 2
NS = 16
NW = NC * NS
CHUNK = 128
BN = 1000


def _mesh():
    return plsc.VectorSubcoreMesh(core_axis_name="c", subcore_axis_name="s")


_SC_PARAMS = pltpu.CompilerParams(use_tc_tiling_on_sc=False)


STG = 200
STG2 = 125
zb1 = 208
NBUF = 4


def _make_deg(n, e_pad):
    ch = e_pad // (NW * CHUNK)
    nco = n // BN

    @functools.partial(
        pl.kernel,
        out_type=jax.ShapeDtypeStruct((NC * n,), jnp.float32),
        mesh=_mesh(),
        compiler_params=_SC_PARAMS,
        scratch_types=[
            pltpu.VMEM((ch, CHUNK), jnp.int32),
            pltpu.VMEM((CHUNK,), jnp.float32),
            pltpu.VMEM((zb1,), jnp.float32),
            pltpu.VMEM_SHARED((n + 16,), jnp.float32),
            pltpu.SemaphoreType.DMA,
        ],
    )
    def deg_k(dst2_hbm, out_hbm, didx2, ones_v, zbuf, acc, sem):
        cid = lax.axis_index("c")
        sid = lax.axis_index("s")
        gid = cid * NS + sid
        pltpu.sync_copy(dst2_hbm.at[pl.ds(gid * ch, ch), :], didx2)

        for k in range(CHUNK // 16):
            ones_v[pl.ds(k * 16, 16)] = jnp.ones((16,), jnp.float32)
        for j in range(zb1 // 16):
            zbuf[pl.ds(j * 16, 16)] = jnp.zeros((16,), jnp.float32)

        @pl.when(sid < nco)
        def _():
            for k in range(BN // STG):
                pltpu.sync_copy(zbuf.at[pl.ds(0, STG)],
                                acc.at[pl.ds(sid * BN + k * STG, STG)])

        @pl.when(sid == nco)
        def _():
            pltpu.sync_copy(zbuf.at[pl.ds(0, 16)], acc.at[pl.ds(n, 16)])

        plsc.subcore_barrier()

        def fire(j, carry):
            pltpu.async_copy(ones_v, acc.at[didx2.at[j]], sem, add=True)
            return carry

        lax.fori_loop(0, ch, fire, 0)

        def drain(j, carry):
            pltpu.make_async_copy(ones_v, acc.at[pl.ds(0, CHUNK)], sem).wait()
            return carry

        lax.fori_loop(0, ch, drain, 0)
        plsc.subcore_barrier()

        @pl.when(sid < nco)
        def _():
            for k in range(BN // STG):
                r = sid * BN + k * STG
                pltpu.sync_copy(acc.at[pl.ds(r, STG)], zbuf.at[pl.ds(0, STG)])
                pltpu.sync_copy(zbuf.at[pl.ds(0, STG)],
                                out_hbm.at[pl.ds(cid * n + r, STG)])

    return deg_k


def _make_prop(n, e_pad, w):
    ch = e_pad // (NW * CHUNK)
    assert ch % NBUF == 0
    nco = n // BN

    @functools.partial(
        pl.kernel,
        out_type=jax.ShapeDtypeStruct((NC * n, w), jnp.float32),
        mesh=_mesh(),
        compiler_params=_SC_PARAMS,
        scratch_types=[
            pltpu.VMEM((ch, CHUNK), jnp.int32),
            pltpu.VMEM((ch, CHUNK), jnp.int32),
            pltpu.VMEM((NBUF, CHUNK, w), jnp.float32),
            pltpu.VMEM_SHARED((n + 8, w), jnp.float32),
        ] + [pltpu.SemaphoreType.DMA] * (2 * NBUF),
    )
    def prop_k(tab_hbm, src2_hbm, dst2_hbm, out_hbm,
               sidx2, didx2, rows, acc, *sems):
        gsem = sems[:NBUF]
        ssem = sems[NBUF:]
        cid = lax.axis_index("c")
        sid = lax.axis_index("s")
        gid = cid * NS + sid
        pltpu.sync_copy(src2_hbm.at[pl.ds(gid * ch, ch), :], sidx2)
        pltpu.sync_copy(dst2_hbm.at[pl.ds(gid * ch, ch), :], didx2)

        def zfill(i, carry):
            for j in range(w // 16):
                rows[0, i, pl.ds(j * 16, 16)] = jnp.zeros((16,), jnp.float32)
            return carry

        lax.fori_loop(0, CHUNK, zfill, 0)

        @pl.when(sid < nco)
        def _():
            for k in range(BN // STG2):
                pltpu.sync_copy(rows.at[0, pl.ds(0, STG2), :],
                                acc.at[pl.ds(sid * BN + k * STG2, STG2), :])

        @pl.when(sid == nco)
        def _():
            pltpu.sync_copy(rows.at[0, pl.ds(0, 8), :], acc.at[pl.ds(n, 8), :])

        plsc.subcore_barrier()

        def gather(j, b):
            return pltpu.async_copy(tab_hbm.at[sidx2.at[j]], rows.at[b],
                                    gsem[b])

        def scatter(j, b):
            return pltpu.async_copy(rows.at[b], acc.at[didx2.at[j]],
                                    ssem[b], add=True)

        for b in range(NBUF):
            gather(b, b)

        def group(g, carry):
            base = g * NBUF
            for b in range(NBUF):
                pltpu.make_async_copy(tab_hbm.at[sidx2.at[base + b]],
                                      rows.at[b], gsem[b]).wait()
                scatter(base + b, b)
            for b in range(NBUF):
                j2 = base + NBUF + b
                pltpu.make_async_copy(rows.at[b],
                                      acc.at[pl.ds(0, CHUNK), :],
                                      ssem[b]).wait()

                @pl.when(j2 < ch)
                def _():
                    gather(j2, b)
            return carry

        lax.fori_loop(0, ch // NBUF, group, 0)
        plsc.subcore_barrier()

        @pl.when(sid < nco)
        def _():
            for k in range(BN // STG2):
                r = sid * BN + k * STG2
                b = k % NBUF
                pltpu.sync_copy(acc.at[pl.ds(r, STG2), :],
                                rows.at[b, pl.ds(0, STG2), :])
                pltpu.sync_copy(rows.at[b, pl.ds(0, STG2), :],
                                out_hbm.at[pl.ds(cid * n + r, STG2), :])

    return prop_k


def _mm_body(x_ref, xd_ref, degT_ref, W1_ref, Wd1_ref, up_ref):
    h = W1_ref.shape[1]
    hd = Wd1_ref.shape[1]
    dinv = lax.rsqrt(1.0 + degT_ref[:, 0:1] + degT_ref[:, 1:2])
    a = jnp.dot(x_ref[...], W1_ref[...], preferred_element_type=jnp.float32)
    b = jnp.dot(xd_ref[...], Wd1_ref[...], preferred_element_type=jnp.float32)
    up_ref[:, 0:h] = a * dinv
    up_ref[:, h:h + hd] = b * dinv


def _mid_body(agg0_ref, agg1_ref, up_ref, degT_ref, b1_ref, Wc_ref, z_ref):
    dinv = lax.rsqrt(1.0 + degT_ref[:, 0:1] + degT_ref[:, 1:2])
    v = jnp.maximum(
        dinv * (agg0_ref[...] + agg1_ref[...] + up_ref[...]) + b1_ref[...], 0.0)
    z_ref[...] = dinv * jnp.dot(v, Wc_ref[...],
                                preferred_element_type=jnp.float32)


def _out_body(agg0_ref, agg1_ref, z_ref, degT_ref, bc_ref, o_ref):
    c = o_ref.shape[1]
    dinv = lax.rsqrt(1.0 + degT_ref[:, 0:1] + degT_ref[:, 1:2])
    pre = dinv * (agg0_ref[...] + agg1_ref[...] + z_ref[...]) + bc_ref[...]
    col = lax.broadcasted_iota(jnp.int32, pre.shape, 1)
    prem = jnp.where(col < c, pre, -1e30)
    m = jnp.max(prem, axis=1, keepdims=True)
    ex = jnp.where(col < c, jnp.exp(prem - m), 0.0)
    lse = jnp.log(jnp.sum(ex, axis=1, keepdims=True)) + m
    o_ref[...] = (pre - lse)[:, 0:c]


def _row_spec(w):
    return pl.BlockSpec((BN, w), lambda i: (i, 0))


def _full_spec(a, b):
    return pl.BlockSpec((a, b), lambda i: (0, 0))


def kernel(x, x_deepwalk, edge_index, W1, b1, W2, b2, Wd1, bd1, Wd2, bd2):
    n, d = x.shape
    dw = x_deepwalk.shape[1]
    e = edge_index.shape[1]
    h = W1.shape[1]
    hd = Wd1.shape[1]
    c = W2.shape[1]
    wu = h + hd
    wz = 16
    f32 = jnp.float32

    ch = -(-e // (NW * CHUNK))
    ch = -(-ch // NBUF) * NBUF
    e_pad = ch * NW * CHUNK
    fill = jnp.full((e_pad - e,), n, jnp.int32)
    srcp = jnp.concatenate([edge_index[0], fill]).reshape(e_pad // CHUNK, CHUNK)
    dstp = jnp.concatenate([edge_index[1], fill]).reshape(e_pad // CHUNK, CHUNK)

    b1cat = jnp.concatenate([b1, bd1]).reshape(1, wu)
    wc = jnp.concatenate([0.2 * W2, 0.1 * Wd2], axis=0)
    wc16 = jnp.pad(wc, ((0, 0), (0, wz - c)))
    bc16 = jnp.pad(0.2 * b2 + 0.1 * bd2, (0, wz - c)).reshape(1, wz)

    deg2 = _make_deg(n, e_pad)(dstp)
    degT = jnp.stack([deg2[:n], deg2[n:]], axis=1)

    grid = n // BN
    up = pl.pallas_call(
        _mm_body,
        grid=(grid,),
        in_specs=[_row_spec(d), _row_spec(dw), _row_spec(2),
                  _full_spec(d, h), _full_spec(dw, hd)],
        out_specs=_row_spec(wu),
        out_shape=jax.ShapeDtypeStruct((n, wu), f32),
    )(x, x_deepwalk, degT, W1, Wd1)

    up_pad = jnp.concatenate([up, jnp.zeros((8, wu), f32)])
    agg = _make_prop(n, e_pad, wu)(up_pad, srcp, dstp)

    z16 = pl.pallas_call(
        _mid_body,
        grid=(grid,),
        in_specs=[_row_spec(wu), _row_spec(wu), _row_spec(wu), _row_spec(2),
                  _full_spec(1, wu), _full_spec(wu, wz)],
        out_specs=_row_spec(wz),
        out_shape=jax.ShapeDtypeStruct((n, wz), f32),
    )(agg[:n], agg[n:], up, degT, b1cat, wc16)

    z_pad = jnp.concatenate([z16, jnp.zeros((8, wz), f32)])
    agg2 = _make_prop(n, e_pad, wz)(z_pad, srcp, dstp)

    out = pl.pallas_call(
        _out_body,
        grid=(grid,),
        in_specs=[_row_spec(wz), _row_spec(wz), _row_spec(wz), _row_spec(2),
                  _full_spec(1, wz)],
        out_specs=_row_spec(c),
        out_shape=jax.ShapeDtypeStruct((n, c), f32),
    )(agg2[:n], agg2[n:], z16, degT, bc16)
    return out


# speedup vs baseline: 37.2267x; 1.8132x over previous
import functools

import jax
import jax.numpy as jnp
from jax import lax
from jax.experimental import pallas as pl
from jax.experimental.pallas import tpu as pltpu
from jax.experimental.pallas import tpu_sc as plsc

NC = 2
NS = 16
NW = NC * NS
CHUNK = 128
BN = 1000


def _mesh():
    return plsc.VectorSubcoreMesh(core_axis_name="c", subcore_axis_name="s")


_SC_PARAMS = pltpu.CompilerParams(use_tc_tiling_on_sc=False)


STG = 200
STG2 = 125
zb1 = 208
NBUF = 4
PADR = 32


def _make_deg(n, e_pad):
    ch = e_pad // (NW * CHUNK)
    nco = n // BN

    @functools.partial(
        pl.kernel,
        out_type=jax.ShapeDtypeStruct((NC * n,), jnp.float32),
        mesh=_mesh(),
        compiler_params=_SC_PARAMS,
        scratch_types=[
            pltpu.VMEM((ch, CHUNK), jnp.int32),
            pltpu.VMEM((CHUNK,), jnp.float32),
            pltpu.VMEM((zb1,), jnp.float32),
            pltpu.VMEM_SHARED((n + PADR,), jnp.float32),
            pltpu.SemaphoreType.DMA,
        ],
    )
    def deg_k(dst2_hbm, out_hbm, didx2, ones_v, zbuf, acc, sem):
        cid = lax.axis_index("c")
        sid = lax.axis_index("s")
        gid = cid * NS + sid
        pltpu.sync_copy(dst2_hbm.at[pl.ds(gid * ch, ch), :], didx2)

        for k in range(CHUNK // 16):
            ones_v[pl.ds(k * 16, 16)] = jnp.ones((16,), jnp.float32)
        for j in range(zb1 // 16):
            zbuf[pl.ds(j * 16, 16)] = jnp.zeros((16,), jnp.float32)

        @pl.when(sid < nco)
        def _():
            for k in range(BN // STG):
                pltpu.sync_copy(zbuf.at[pl.ds(0, STG)],
                                acc.at[pl.ds(sid * BN + k * STG, STG)])

        @pl.when(sid == nco)
        def _():
            pltpu.sync_copy(zbuf.at[pl.ds(0, PADR)], acc.at[pl.ds(n, PADR)])

        plsc.subcore_barrier()

        def fire(j, carry):
            pltpu.async_copy(ones_v, acc.at[didx2.at[j]], sem, add=True)
            return carry

        lax.fori_loop(0, ch, fire, 0)

        def drain(j, carry):
            pltpu.make_async_copy(ones_v, acc.at[pl.ds(0, CHUNK)], sem).wait()
            return carry

        lax.fori_loop(0, ch, drain, 0)
        plsc.subcore_barrier()

        @pl.when(sid < nco)
        def _():
            for k in range(BN // STG):
                r = sid * BN + k * STG
                pltpu.sync_copy(acc.at[pl.ds(r, STG)], zbuf.at[pl.ds(0, STG)])
                pltpu.sync_copy(zbuf.at[pl.ds(0, STG)],
                                out_hbm.at[pl.ds(cid * n + r, STG)])

    return deg_k


def _make_prop(n, e_pad, w):
    ch = e_pad // (NW * CHUNK)
    assert ch % NBUF == 0
    nco = n // BN

    @functools.partial(
        pl.kernel,
        out_type=jax.ShapeDtypeStruct((NC * n, w), jnp.float32),
        mesh=_mesh(),
        compiler_params=_SC_PARAMS,
        scratch_types=[
            pltpu.VMEM((ch, CHUNK), jnp.int32),
            pltpu.VMEM((ch, CHUNK), jnp.int32),
            pltpu.VMEM((NBUF, CHUNK, w), jnp.float32),
            pltpu.VMEM_SHARED((n + PADR, w), jnp.float32),
        ] + [pltpu.SemaphoreType.DMA] * (2 * NBUF),
    )
    def prop_k(tab_hbm, src2_hbm, dst2_hbm, out_hbm,
               sidx2, didx2, rows, acc, *sems):
        gsem = sems[:NBUF]
        ssem = sems[NBUF:]
        cid = lax.axis_index("c")
        sid = lax.axis_index("s")
        gid = cid * NS + sid
        pltpu.sync_copy(src2_hbm.at[pl.ds(gid * ch, ch), :], sidx2)
        pltpu.sync_copy(dst2_hbm.at[pl.ds(gid * ch, ch), :], didx2)

        def zfill(i, carry):
            for j in range(w // 16):
                rows[0, i, pl.ds(j * 16, 16)] = jnp.zeros((16,), jnp.float32)
            return carry

        lax.fori_loop(0, CHUNK, zfill, 0)

        @pl.when(sid < nco)
        def _():
            for k in range(BN // STG2):
                pltpu.sync_copy(rows.at[0, pl.ds(0, STG2), :],
                                acc.at[pl.ds(sid * BN + k * STG2, STG2), :])

        @pl.when(sid == nco)
        def _():
            pltpu.sync_copy(rows.at[0, pl.ds(0, PADR), :],
                            acc.at[pl.ds(n, PADR), :])

        plsc.subcore_barrier()

        def gather(j, b):
            return pltpu.async_copy(tab_hbm.at[sidx2.at[j]], rows.at[b],
                                    gsem[b])

        def scatter(j, b):
            return pltpu.async_copy(rows.at[b], acc.at[didx2.at[j]],
                                    ssem[b], add=True)

        for b in range(NBUF):
            gather(b, b)

        def group(g, carry):
            base = g * NBUF
            for b in range(NBUF):
                pltpu.make_async_copy(tab_hbm.at[sidx2.at[base + b]],
                                      rows.at[b], gsem[b]).wait()
                scatter(base + b, b)
            for b in range(NBUF):
                j2 = base + NBUF + b
                pltpu.make_async_copy(rows.at[b],
                                      acc.at[pl.ds(0, CHUNK), :],
                                      ssem[b]).wait()

                @pl.when(j2 < ch)
                def _():
                    gather(j2, b)
            return carry

        lax.fori_loop(0, ch // NBUF, group, 0)
        plsc.subcore_barrier()

        @pl.when(sid < nco)
        def _():
            for k in range(BN // STG2):
                r = sid * BN + k * STG2
                b = k % NBUF
                pltpu.sync_copy(acc.at[pl.ds(r, STG2), :],
                                rows.at[b, pl.ds(0, STG2), :])
                pltpu.sync_copy(rows.at[b, pl.ds(0, STG2), :],
                                out_hbm.at[pl.ds(cid * n + r, STG2), :])

    return prop_k


def _mm_body(x_ref, xd_ref, degT_ref, W1_ref, Wd1_ref, up_ref):
    h = W1_ref.shape[1]
    hd = Wd1_ref.shape[1]
    dinv = lax.rsqrt(1.0 + degT_ref[:, 0:1] + degT_ref[:, 1:2])
    a = jnp.dot(x_ref[...], W1_ref[...], preferred_element_type=jnp.float32)
    b = jnp.dot(xd_ref[...], Wd1_ref[...], preferred_element_type=jnp.float32)
    up_ref[:, 0:h] = a * dinv
    up_ref[:, h:h + hd] = b * dinv


def _mid_body(agg0_ref, agg1_ref, up_ref, degT_ref, b1_ref, Wc_ref, z_ref):
    dinv = lax.rsqrt(1.0 + degT_ref[:, 0:1] + degT_ref[:, 1:2])
    v = jnp.maximum(
        dinv * (agg0_ref[...] + agg1_ref[...] + up_ref[...]) + b1_ref[...], 0.0)
    z_ref[...] = dinv * jnp.dot(v, Wc_ref[...],
                                preferred_element_type=jnp.float32)


def _out_body(agg0_ref, agg1_ref, z_ref, degT_ref, bc_ref, o_ref):
    c = o_ref.shape[1]
    dinv = lax.rsqrt(1.0 + degT_ref[:, 0:1] + degT_ref[:, 1:2])
    pre = dinv * (agg0_ref[...] + agg1_ref[...] + z_ref[...]) + bc_ref[...]
    col = lax.broadcasted_iota(jnp.int32, pre.shape, 1)
    prem = jnp.where(col < c, pre, -1e30)
    m = jnp.max(prem, axis=1, keepdims=True)
    ex = jnp.where(col < c, jnp.exp(prem - m), 0.0)
    lse = jnp.log(jnp.sum(ex, axis=1, keepdims=True)) + m
    o_ref[...] = (pre - lse)[:, 0:c]


def _row_spec(w):
    return pl.BlockSpec((BN, w), lambda i: (i, 0))


def _full_spec(a, b):
    return pl.BlockSpec((a, b), lambda i: (0, 0))


def kernel(x, x_deepwalk, edge_index, W1, b1, W2, b2, Wd1, bd1, Wd2, bd2):
    n, d = x.shape
    dw = x_deepwalk.shape[1]
    e = edge_index.shape[1]
    h = W1.shape[1]
    hd = Wd1.shape[1]
    c = W2.shape[1]
    wu = h + hd
    wz = 16
    f32 = jnp.float32

    ch = -(-e // (NW * CHUNK))
    ch = -(-ch // NBUF) * NBUF
    e_pad = ch * NW * CHUNK
    npd = jnp.arange(e_pad - e, dtype=jnp.int32)
    srcp = jnp.concatenate([edge_index[0], n + (npd % 8)])
    dstp = jnp.concatenate([edge_index[1], n + (npd % PADR)])
    srcp = srcp.reshape(e_pad // CHUNK, CHUNK)
    dstp = dstp.reshape(e_pad // CHUNK, CHUNK)

    b1cat = jnp.concatenate([b1, bd1]).reshape(1, wu)
    wc = jnp.concatenate([0.2 * W2, 0.1 * Wd2], axis=0)
    wc16 = jnp.pad(wc, ((0, 0), (0, wz - c)))
    bc16 = jnp.pad(0.2 * b2 + 0.1 * bd2, (0, wz - c)).reshape(1, wz)

    deg2 = _make_deg(n, e_pad)(dstp)
    degT = jnp.stack([deg2[:n], deg2[n:]], axis=1)

    grid = n // BN
    up = pl.pallas_call(
        _mm_body,
        grid=(grid,),
        in_specs=[_row_spec(d), _row_spec(dw), _row_spec(2),
                  _full_spec(d, h), _full_spec(dw, hd)],
        out_specs=_row_spec(wu),
        out_shape=jax.ShapeDtypeStruct((n, wu), f32),
    )(x, x_deepwalk, degT, W1, Wd1)

    up_pad = jnp.concatenate([up, jnp.zeros((8, wu), f32)])
    agg = _make_prop(n, e_pad, wu)(up_pad, srcp, dstp)

    z16 = pl.pallas_call(
        _mid_body,
        grid=(grid,),
        in_specs=[_row_spec(wu), _row_spec(wu), _row_spec(wu), _row_spec(2),
                  _full_spec(1, wu), _full_spec(wu, wz)],
        out_specs=_row_spec(wz),
        out_shape=jax.ShapeDtypeStruct((n, wz), f32),
    )(agg[:n], agg[n:], up, degT, b1cat, wc16)

    z_pad = jnp.concatenate([z16, jnp.zeros((8, wz), f32)])
    agg2 = _make_prop(n, e_pad, wz)(z_pad, srcp, dstp)

    out = pl.pallas_call(
        _out_body,
        grid=(grid,),
        in_specs=[_row_spec(wz), _row_spec(wz), _row_spec(wz), _row_spec(2),
                  _full_spec(1, wz)],
        out_specs=_row_spec(c),
        out_shape=jax.ShapeDtypeStruct((n, c), f32),
    )(agg2[:n], agg2[n:], z16, degT, bc16)
    return out


# speedup vs baseline: 39.7179x; 1.0669x over previous
import functools

import jax
import jax.numpy as jnp
from jax import lax
from jax.experimental import pallas as pl
from jax.experimental.pallas import tpu as pltpu
from jax.experimental.pallas import tpu_sc as plsc

NC = 2
NS = 16
NW = NC * NS
CHUNK = 128
BN = 1000


def _mesh():
    return plsc.VectorSubcoreMesh(core_axis_name="c", subcore_axis_name="s")


_SC_PARAMS = pltpu.CompilerParams(use_tc_tiling_on_sc=False)


STG = 200
STG2 = 125
zb1 = 208
NBUF = 4
PADR = 32


def _make_deg(n, e_pad):
    ch = e_pad // (NW * CHUNK)
    nco = n // BN

    @functools.partial(
        pl.kernel,
        out_type=jax.ShapeDtypeStruct((NC * n,), jnp.float32),
        mesh=_mesh(),
        compiler_params=_SC_PARAMS,
        scratch_types=[
            pltpu.VMEM((ch, CHUNK), jnp.int32),
            pltpu.VMEM((CHUNK,), jnp.float32),
            pltpu.VMEM((zb1,), jnp.float32),
            pltpu.VMEM_SHARED((n + PADR,), jnp.float32),
            pltpu.SemaphoreType.DMA,
        ],
    )
    def deg_k(dst2_hbm, out_hbm, didx2, ones_v, zbuf, acc, sem):
        cid = lax.axis_index("c")
        sid = lax.axis_index("s")
        gid = cid * NS + sid
        pltpu.sync_copy(dst2_hbm.at[pl.ds(gid * ch, ch), :], didx2)

        for k in range(CHUNK // 16):
            ones_v[pl.ds(k * 16, 16)] = jnp.ones((16,), jnp.float32)
        for j in range(zb1 // 16):
            zbuf[pl.ds(j * 16, 16)] = jnp.zeros((16,), jnp.float32)

        @pl.when(sid < nco)
        def _():
            for k in range(BN // STG):
                pltpu.sync_copy(zbuf.at[pl.ds(0, STG)],
                                acc.at[pl.ds(sid * BN + k * STG, STG)])

        @pl.when(sid == nco)
        def _():
            pltpu.sync_copy(zbuf.at[pl.ds(0, PADR)], acc.at[pl.ds(n, PADR)])

        plsc.subcore_barrier()

        def fire(j, carry):
            pltpu.async_copy(ones_v, acc.at[didx2.at[j]], sem, add=True)
            return carry

        lax.fori_loop(0, ch, fire, 0)

        def drain(j, carry):
            pltpu.make_async_copy(ones_v, acc.at[pl.ds(0, CHUNK)], sem).wait()
            return carry

        lax.fori_loop(0, ch, drain, 0)
        plsc.subcore_barrier()

        @pl.when(sid < nco)
        def _():
            for k in range(BN // STG):
                r = sid * BN + k * STG
                pltpu.sync_copy(acc.at[pl.ds(r, STG)], zbuf.at[pl.ds(0, STG)])
                pltpu.sync_copy(zbuf.at[pl.ds(0, STG)],
                                out_hbm.at[pl.ds(cid * n + r, STG)])

    return deg_k


def _make_prop(n, e_pad, w):
    ch = e_pad // (NW * CHUNK)
    assert ch % NBUF == 0
    nco = n // BN

    @functools.partial(
        pl.kernel,
        out_type=jax.ShapeDtypeStruct((NC * n, w), jnp.float32),
        mesh=_mesh(),
        compiler_params=_SC_PARAMS,
        scratch_types=[
            pltpu.VMEM((ch, CHUNK), jnp.int32),
            pltpu.VMEM((ch, CHUNK), jnp.int32),
            pltpu.VMEM((NBUF, CHUNK, w), jnp.float32),
            pltpu.VMEM_SHARED((n + PADR, w), jnp.float32),
        ] + [pltpu.SemaphoreType.DMA] * (2 * NBUF),
    )
    def prop_k(tab_hbm, src2_hbm, dst2_hbm, out_hbm,
               sidx2, didx2, rows, acc, *sems):
        gsem = sems[:NBUF]
        ssem = sems[NBUF:]
        cid = lax.axis_index("c")
        sid = lax.axis_index("s")
        gid = cid * NS + sid
        pltpu.sync_copy(src2_hbm.at[pl.ds(gid * ch, ch), :], sidx2)
        pltpu.sync_copy(dst2_hbm.at[pl.ds(gid * ch, ch), :], didx2)

        def zfill(i, carry):
            for j in range(w // 16):
                rows[0, i, pl.ds(j * 16, 16)] = jnp.zeros((16,), jnp.float32)
            return carry

        lax.fori_loop(0, CHUNK, zfill, 0)

        @pl.when(sid < nco)
        def _():
            for k in range(BN // STG2):
                pltpu.sync_copy(rows.at[0, pl.ds(0, STG2), :],
                                acc.at[pl.ds(sid * BN + k * STG2, STG2), :])

        @pl.when(sid == nco)
        def _():
            pltpu.sync_copy(rows.at[0, pl.ds(0, PADR), :],
                            acc.at[pl.ds(n, PADR), :])

        plsc.subcore_barrier()

        def gather(j, b):
            return pltpu.async_copy(tab_hbm.at[sidx2.at[j]], rows.at[b],
                                    gsem[b])

        def scatter(j, b):
            return pltpu.async_copy(rows.at[b], acc.at[didx2.at[j]],
                                    ssem[b], add=True)

        for b in range(NBUF):
            gather(b, b)

        def group(g, carry):
            base = g * NBUF
            for b in range(NBUF):
                pltpu.make_async_copy(tab_hbm.at[sidx2.at[base + b]],
                                      rows.at[b], gsem[b]).wait()
                scatter(base + b, b)
            for b in range(NBUF):
                j2 = base + NBUF + b
                pltpu.make_async_copy(rows.at[b],
                                      acc.at[pl.ds(0, CHUNK), :],
                                      ssem[b]).wait()

                @pl.when(j2 < ch)
                def _():
                    gather(j2, b)
            return carry

        lax.fori_loop(0, ch // NBUF, group, 0)
        plsc.subcore_barrier()

        @pl.when(sid < nco)
        def _():
            for k in range(BN // STG2):
                r = sid * BN + k * STG2
                b = k % NBUF
                pltpu.sync_copy(acc.at[pl.ds(r, STG2), :],
                                rows.at[b, pl.ds(0, STG2), :])
                pltpu.sync_copy(rows.at[b, pl.ds(0, STG2), :],
                                out_hbm.at[pl.ds(cid * n + r, STG2), :])

    return prop_k


def _mm_body(x_ref, xd_ref, deg0_ref, deg1_ref, W1_ref, Wd1_ref, up_ref):
    h = W1_ref.shape[1]
    hd = Wd1_ref.shape[1]
    dinv = lax.rsqrt(1.0 + deg0_ref[...] + deg1_ref[...])
    a = jnp.dot(x_ref[...], W1_ref[...], preferred_element_type=jnp.float32)
    b = jnp.dot(xd_ref[...], Wd1_ref[...], preferred_element_type=jnp.float32)
    up_ref[:, 0:h] = a * dinv
    up_ref[:, h:h + hd] = b * dinv


def _mid_body(agg0_ref, agg1_ref, up_ref, deg0_ref, deg1_ref, b1_ref, Wc_ref,
              z_ref):
    dinv = lax.rsqrt(1.0 + deg0_ref[...] + deg1_ref[...])
    v = jnp.maximum(
        dinv * (agg0_ref[...] + agg1_ref[...] + up_ref[...]) + b1_ref[...], 0.0)
    z_ref[...] = dinv * jnp.dot(v, Wc_ref[...],
                                preferred_element_type=jnp.float32)


def _out_body(agg0_ref, agg1_ref, z_ref, deg0_ref, deg1_ref, bc_ref, o_ref):
    c = o_ref.shape[1]
    dinv = lax.rsqrt(1.0 + deg0_ref[...] + deg1_ref[...])
    pre = dinv * (agg0_ref[...] + agg1_ref[...] + z_ref[...]) + bc_ref[...]
    col = lax.broadcasted_iota(jnp.int32, pre.shape, 1)
    prem = jnp.where(col < c, pre, -1e30)
    m = jnp.max(prem, axis=1, keepdims=True)
    ex = jnp.where(col < c, jnp.exp(prem - m), 0.0)
    lse = jnp.log(jnp.sum(ex, axis=1, keepdims=True)) + m
    o_ref[...] = (pre - lse)[:, 0:c]


def _row_spec(w):
    return pl.BlockSpec((BN, w), lambda i: (i, 0))


def _full_spec(a, b):
    return pl.BlockSpec((a, b), lambda i: (0, 0))


def kernel(x, x_deepwalk, edge_index, W1, b1, W2, b2, Wd1, bd1, Wd2, bd2):
    n, d = x.shape
    dw = x_deepwalk.shape[1]
    e = edge_index.shape[1]
    h = W1.shape[1]
    hd = Wd1.shape[1]
    c = W2.shape[1]
    wu = h + hd
    wz = 16
    f32 = jnp.float32

    ch = -(-e // (NW * CHUNK))
    ch = -(-ch // NBUF) * NBUF
    e_pad = ch * NW * CHUNK
    npad = e_pad - e
    npd = jnp.arange(npad, dtype=jnp.int32)
    pad_src = npd % 8
    pad_dst = n + (npd % PADR)
    if e % NW == 0 and npad % NW == 0:
        srcp = jnp.concatenate(
            [edge_index[0].reshape(NW, e // NW),
             pad_src.reshape(NW, npad // NW)], axis=1)
        dstp = jnp.concatenate(
            [edge_index[1].reshape(NW, e // NW),
             pad_dst.reshape(NW, npad // NW)], axis=1)
    else:
        srcp = jnp.concatenate([edge_index[0], pad_src])
        dstp = jnp.concatenate([edge_index[1], pad_dst])
    srcp = srcp.reshape(e_pad // CHUNK, CHUNK)
    dstp = dstp.reshape(e_pad // CHUNK, CHUNK)

    b1cat = jnp.concatenate([b1, bd1]).reshape(1, wu)
    wc = jnp.concatenate([0.2 * W2, 0.1 * Wd2], axis=0)
    wc16 = jnp.pad(wc, ((0, 0), (0, wz - c)))
    bc16 = jnp.pad(0.2 * b2 + 0.1 * bd2, (0, wz - c)).reshape(1, wz)

    grid = n // BN

    def _hi_spec(w):
        return pl.BlockSpec((BN, w), lambda i: (i + grid, 0))

    deg2 = _make_deg(n, e_pad)(dstp).reshape(NC * n, 1)

    up = pl.pallas_call(
        _mm_body,
        grid=(grid,),
        in_specs=[_row_spec(d), _row_spec(dw), _row_spec(1), _hi_spec(1),
                  _full_spec(d, h), _full_spec(dw, hd)],
        out_specs=_row_spec(wu),
        out_shape=jax.ShapeDtypeStruct((n, wu), f32),
    )(x, x_deepwalk, deg2, deg2, W1, Wd1)

    agg = _make_prop(n, e_pad, wu)(up, srcp, dstp)

    z16 = pl.pallas_call(
        _mid_body,
        grid=(grid,),
        in_specs=[_row_spec(wu), _hi_spec(wu), _row_spec(wu),
                  _row_spec(1), _hi_spec(1),
                  _full_spec(1, wu), _full_spec(wu, wz)],
        out_specs=_row_spec(wz),
        out_shape=jax.ShapeDtypeStruct((n, wz), f32),
    )(agg, agg, up, deg2, deg2, b1cat, wc16)

    agg2 = _make_prop(n, e_pad, wz)(z16, srcp, dstp)

    out = pl.pallas_call(
        _out_body,
        grid=(grid,),
        in_specs=[_row_spec(wz), _hi_spec(wz), _row_spec(wz),
                  _row_spec(1), _hi_spec(1),
                  _full_spec(1, wz)],
        out_specs=_row_spec(c),
        out_shape=jax.ShapeDtypeStruct((n, c), f32),
    )(agg2, agg2, z16, deg2, deg2, bc16)
    return out


# speedup vs baseline: 49.9466x; 1.2575x over previous
import functools

import jax
import jax.numpy as jnp
from jax import lax
from jax.experimental import pallas as pl
from jax.experimental.pallas import tpu as pltpu
from jax.experimental.pallas import tpu_sc as plsc

NC = 2
NS = 16
NW = NC * NS
CHUNK = 128
BN = 1000


def _mesh():
    return plsc.VectorSubcoreMesh(core_axis_name="c", subcore_axis_name="s")


_SC_PARAMS = pltpu.CompilerParams(use_tc_tiling_on_sc=False)


STG = 200
STG2 = 125
zb1 = 208
NBUF = 4
PADR = 128


def _make_deg(n, e_pad):
    ch = e_pad // (NW * CHUNK)
    nco = n // BN

    @functools.partial(
        pl.kernel,
        out_type=jax.ShapeDtypeStruct((NC * n,), jnp.float32),
        mesh=_mesh(),
        compiler_params=_SC_PARAMS,
        scratch_types=[
            pltpu.VMEM((ch, CHUNK), jnp.int32),
            pltpu.VMEM((CHUNK,), jnp.float32),
            pltpu.VMEM((zb1,), jnp.float32),
            pltpu.VMEM_SHARED((n + PADR,), jnp.float32),
            pltpu.SemaphoreType.DMA,
        ],
    )
    def deg_k(dst2_hbm, out_hbm, didx2, ones_v, zbuf, acc, sem):
        cid = lax.axis_index("c")
        sid = lax.axis_index("s")
        gid = cid * NS + sid
        pltpu.sync_copy(dst2_hbm.at[pl.ds(gid * ch, ch), :], didx2)

        for k in range(CHUNK // 16):
            ones_v[pl.ds(k * 16, 16)] = jnp.ones((16,), jnp.float32)
        for j in range(zb1 // 16):
            zbuf[pl.ds(j * 16, 16)] = jnp.zeros((16,), jnp.float32)

        @pl.when(sid < nco)
        def _():
            for k in range(BN // STG):
                pltpu.sync_copy(zbuf.at[pl.ds(0, STG)],
                                acc.at[pl.ds(sid * BN + k * STG, STG)])

        @pl.when(sid == nco)
        def _():
            pltpu.sync_copy(zbuf.at[pl.ds(0, PADR)], acc.at[pl.ds(n, PADR)])

        plsc.subcore_barrier()

        def fire(j, carry):
            pltpu.async_copy(ones_v, acc.at[didx2.at[j]], sem, add=True)
            return carry

        lax.fori_loop(0, ch, fire, 0)

        def drain(j, carry):
            pltpu.make_async_copy(ones_v, acc.at[pl.ds(0, CHUNK)], sem).wait()
            return carry

        lax.fori_loop(0, ch, drain, 0)
        plsc.subcore_barrier()

        @pl.when(sid < nco)
        def _():
            for k in range(BN // STG):
                r = sid * BN + k * STG
                pltpu.sync_copy(acc.at[pl.ds(r, STG)], zbuf.at[pl.ds(0, STG)])
                pltpu.sync_copy(zbuf.at[pl.ds(0, STG)],
                                out_hbm.at[pl.ds(cid * n + r, STG)])

    return deg_k


def _make_prop(n, e_pad, w):
    ch = e_pad // (NW * CHUNK)
    assert ch % NBUF == 0
    nco = n // BN

    @functools.partial(
        pl.kernel,
        out_type=jax.ShapeDtypeStruct((NC * n, w), jnp.float32),
        mesh=_mesh(),
        compiler_params=_SC_PARAMS,
        scratch_types=[
            pltpu.VMEM((ch, CHUNK), jnp.int32),
            pltpu.VMEM((ch, CHUNK), jnp.int32),
            pltpu.VMEM((NBUF, CHUNK, w), jnp.float32),
            pltpu.VMEM_SHARED((n + PADR, w), jnp.float32),
        ] + [pltpu.SemaphoreType.DMA] * (2 * NBUF),
    )
    def prop_k(tab_hbm, src2_hbm, dst2_hbm, out_hbm,
               sidx2, didx2, rows, acc, *sems):
        gsem = sems[:NBUF]
        ssem = sems[NBUF:]
        cid = lax.axis_index("c")
        sid = lax.axis_index("s")
        gid = cid * NS + sid
        pltpu.sync_copy(src2_hbm.at[pl.ds(gid * ch, ch), :], sidx2)
        pltpu.sync_copy(dst2_hbm.at[pl.ds(gid * ch, ch), :], didx2)

        def zfill(i, carry):
            for j in range(w // 16):
                rows[0, i, pl.ds(j * 16, 16)] = jnp.zeros((16,), jnp.float32)
            return carry

        lax.fori_loop(0, CHUNK, zfill, 0)

        @pl.when(sid < nco)
        def _():
            for k in range(BN // STG2):
                pltpu.sync_copy(rows.at[0, pl.ds(0, STG2), :],
                                acc.at[pl.ds(sid * BN + k * STG2, STG2), :])

        @pl.when(sid == nco)
        def _():
            pltpu.sync_copy(rows.at[0, pl.ds(0, PADR), :],
                            acc.at[pl.ds(n, PADR), :])

        plsc.subcore_barrier()

        def gather(j, b):
            return pltpu.async_copy(tab_hbm.at[sidx2.at[j]], rows.at[b],
                                    gsem[b])

        def scatter(j, b):
            return pltpu.async_copy(rows.at[b], acc.at[didx2.at[j]],
                                    ssem[b], add=True)

        for b in range(NBUF):
            gather(b, b)

        def group(g, carry):
            base = g * NBUF
            for b in range(NBUF):
                pltpu.make_async_copy(tab_hbm.at[sidx2.at[base + b]],
                                      rows.at[b], gsem[b]).wait()
                scatter(base + b, b)
            for b in range(NBUF):
                j2 = base + NBUF + b
                pltpu.make_async_copy(rows.at[b],
                                      acc.at[pl.ds(0, CHUNK), :],
                                      ssem[b]).wait()

                @pl.when(j2 < ch)
                def _():
                    gather(j2, b)
            return carry

        lax.fori_loop(0, ch // NBUF, group, 0)
        plsc.subcore_barrier()

        @pl.when(sid < nco)
        def _():
            for k in range(BN // STG2):
                r = sid * BN + k * STG2
                b = k % NBUF
                pltpu.sync_copy(acc.at[pl.ds(r, STG2), :],
                                rows.at[b, pl.ds(0, STG2), :])
                pltpu.sync_copy(rows.at[b, pl.ds(0, STG2), :],
                                out_hbm.at[pl.ds(cid * n + r, STG2), :])

    return prop_k


def _mm_body(x_ref, xd_ref, deg0_ref, deg1_ref, W1_ref, Wd1_ref, up_ref):
    h = W1_ref.shape[1]
    hd = Wd1_ref.shape[1]
    dinv = lax.rsqrt(1.0 + deg0_ref[...] + deg1_ref[...])
    a = jnp.dot(x_ref[...], W1_ref[...], preferred_element_type=jnp.float32)
    b = jnp.dot(xd_ref[...], Wd1_ref[...], preferred_element_type=jnp.float32)
    up_ref[:, 0:h] = a * dinv
    up_ref[:, h:h + hd] = b * dinv


def _mid_body(agg0_ref, agg1_ref, up_ref, deg0_ref, deg1_ref, b1_ref, Wc_ref,
              z_ref):
    dinv = lax.rsqrt(1.0 + deg0_ref[...] + deg1_ref[...])
    v = jnp.maximum(
        dinv * (agg0_ref[...] + agg1_ref[...] + up_ref[...]) + b1_ref[...], 0.0)
    z_ref[...] = dinv * jnp.dot(v, Wc_ref[...],
                                preferred_element_type=jnp.float32)


def _out_body(agg0_ref, agg1_ref, z_ref, deg0_ref, deg1_ref, bc_ref, o_ref):
    c = o_ref.shape[1]
    dinv = lax.rsqrt(1.0 + deg0_ref[...] + deg1_ref[...])
    pre = dinv * (agg0_ref[...] + agg1_ref[...] + z_ref[...]) + bc_ref[...]
    col = lax.broadcasted_iota(jnp.int32, pre.shape, 1)
    prem = jnp.where(col < c, pre, -1e30)
    m = jnp.max(prem, axis=1, keepdims=True)
    ex = jnp.where(col < c, jnp.exp(prem - m), 0.0)
    lse = jnp.log(jnp.sum(ex, axis=1, keepdims=True)) + m
    o_ref[...] = (pre - lse)[:, 0:c]


def _row_spec(w):
    return pl.BlockSpec((BN, w), lambda i: (i, 0))


def _full_spec(a, b):
    return pl.BlockSpec((a, b), lambda i: (0, 0))


def kernel(x, x_deepwalk, edge_index, W1, b1, W2, b2, Wd1, bd1, Wd2, bd2):
    n, d = x.shape
    dw = x_deepwalk.shape[1]
    e = edge_index.shape[1]
    h = W1.shape[1]
    hd = Wd1.shape[1]
    c = W2.shape[1]
    wu = h + hd
    wz = 16
    f32 = jnp.float32

    ch = -(-e // (NW * CHUNK))
    ch = -(-ch // NBUF) * NBUF
    e_pad = ch * NW * CHUNK
    npad = e_pad - e
    npd = jnp.arange(npad, dtype=jnp.int32)
    pad_src = npd % CHUNK
    pad_dst = n + (npd % PADR)
    if e % NW == 0 and npad % NW == 0:
        srcp = jnp.concatenate(
            [edge_index[0].reshape(NW, e // NW),
             pad_src.reshape(NW, npad // NW)], axis=1)
        dstp = jnp.concatenate(
            [edge_index[1].reshape(NW, e // NW),
             pad_dst.reshape(NW, npad // NW)], axis=1)
    else:
        srcp = jnp.concatenate([edge_index[0], pad_src])
        dstp = jnp.concatenate([edge_index[1], pad_dst])
    srcp = srcp.reshape(e_pad // CHUNK, CHUNK)
    dstp = dstp.reshape(e_pad // CHUNK, CHUNK)

    b1cat = jnp.concatenate([b1, bd1]).reshape(1, wu)
    wc = jnp.concatenate([0.2 * W2, 0.1 * Wd2], axis=0)
    wc16 = jnp.pad(wc, ((0, 0), (0, wz - c)))
    bc16 = jnp.pad(0.2 * b2 + 0.1 * bd2, (0, wz - c)).reshape(1, wz)

    grid = n // BN

    def _hi_spec(w):
        return pl.BlockSpec((BN, w), lambda i: (i + grid, 0))

    deg2 = _make_deg(n, e_pad)(dstp).reshape(NC * n, 1)

    up = pl.pallas_call(
        _mm_body,
        grid=(grid,),
        in_specs=[_row_spec(d), _row_spec(dw), _row_spec(1), _hi_spec(1),
                  _full_spec(d, h), _full_spec(dw, hd)],
        out_specs=_row_spec(wu),
        out_shape=jax.ShapeDtypeStruct((n, wu), f32),
    )(x, x_deepwalk, deg2, deg2, W1, Wd1)

    agg = _make_prop(n, e_pad, wu)(up, srcp, dstp)

    z16 = pl.pallas_call(
        _mid_body,
        grid=(grid,),
        in_specs=[_row_spec(wu), _hi_spec(wu), _row_spec(wu),
                  _row_spec(1), _hi_spec(1),
                  _full_spec(1, wu), _full_spec(wu, wz)],
        out_specs=_row_spec(wz),
        out_shape=jax.ShapeDtypeStruct((n, wz), f32),
    )(agg, agg, up, deg2, deg2, b1cat, wc16)

    agg2 = _make_prop(n, e_pad, wz)(z16, srcp, dstp)

    out = pl.pallas_call(
        _out_body,
        grid=(grid,),
        in_specs=[_row_spec(wz), _hi_spec(wz), _row_spec(wz),
                  _row_spec(1), _hi_spec(1),
                  _full_spec(1, wz)],
        out_specs=_row_spec(c),
        out_shape=jax.ShapeDtypeStruct((n, c), f32),
    )(agg2, agg2, z16, deg2, deg2, bc16)
    return out


# speedup vs baseline: 54.1327x; 1.0838x over previous
import functools

import jax
import jax.numpy as jnp
from jax import lax
from jax.experimental import pallas as pl
from jax.experimental.pallas import tpu as pltpu
from jax.experimental.pallas import tpu_sc as plsc

NC = 2
NS = 16
NW = NC * NS
CHUNK = 128
BN = 1000
TBN = 2000


def _mesh():
    return plsc.VectorSubcoreMesh(core_axis_name="c", subcore_axis_name="s")


_SC_PARAMS = pltpu.CompilerParams(use_tc_tiling_on_sc=False)


STG = 200
STG2 = 125
zb1 = 208
NBUF = 4
PADR = 128


def _make_deg(n, nrows):
    chb = nrows // NW
    xtra = nrows - chb * NW
    chm = chb + (1 if xtra else 0)
    nco = n // BN

    @functools.partial(
        pl.kernel,
        out_type=jax.ShapeDtypeStruct((NC * n,), jnp.float32),
        mesh=_mesh(),
        compiler_params=_SC_PARAMS,
        scratch_types=[
            pltpu.VMEM((chm, CHUNK), jnp.int32),
            pltpu.VMEM((CHUNK,), jnp.float32),
            pltpu.VMEM((zb1,), jnp.float32),
            pltpu.VMEM_SHARED((n + PADR,), jnp.float32),
            pltpu.SemaphoreType.DMA,
        ],
    )
    def deg_k(ei_hbm, out_hbm, didx2, ones_v, zbuf, acc, sem):
        cid = lax.axis_index("c")
        sid = lax.axis_index("s")
        gid = cid * NS + sid
        ch = chb + jnp.where(gid < xtra, 1, 0)
        off = chb * gid + jnp.minimum(gid, xtra)
        pltpu.sync_copy(ei_hbm.at[1, pl.ds(off, chb), :],
                        didx2.at[pl.ds(0, chb), :])
        if xtra:
            @pl.when(gid < xtra)
            def _():
                pltpu.sync_copy(ei_hbm.at[1, pl.ds(off + chb, 1), :],
                                didx2.at[pl.ds(chb, 1), :])

        for k in range(CHUNK // 16):
            ones_v[pl.ds(k * 16, 16)] = jnp.ones((16,), jnp.float32)
        for j in range(zb1 // 16):
            zbuf[pl.ds(j * 16, 16)] = jnp.zeros((16,), jnp.float32)

        @pl.when(sid < nco)
        def _():
            for k in range(BN // STG):
                pltpu.sync_copy(zbuf.at[pl.ds(0, STG)],
                                acc.at[pl.ds(sid * BN + k * STG, STG)])

        @pl.when(sid == nco)
        def _():
            pltpu.sync_copy(zbuf.at[pl.ds(0, PADR)], acc.at[pl.ds(n, PADR)])

        plsc.subcore_barrier()

        def fire(j, carry):
            pltpu.async_copy(ones_v, acc.at[didx2.at[j]], sem, add=True)
            return carry

        lax.fori_loop(0, ch, fire, 0)

        def drain(j, carry):
            pltpu.make_async_copy(ones_v, acc.at[pl.ds(0, CHUNK)], sem).wait()
            return carry

        lax.fori_loop(0, ch, drain, 0)
        plsc.subcore_barrier()

        @pl.when(sid < nco)
        def _():
            for k in range(BN // STG):
                r = sid * BN + k * STG
                pltpu.sync_copy(acc.at[pl.ds(r, STG)], zbuf.at[pl.ds(0, STG)])
                pltpu.sync_copy(zbuf.at[pl.ds(0, STG)],
                                out_hbm.at[pl.ds(cid * n + r, STG)])

    return deg_k


def _make_prop(n, nrows, w):
    chb = nrows // NW
    xtra = nrows - chb * NW
    chm = chb + (1 if xtra else 0)
    nco = n // BN

    @functools.partial(
        pl.kernel,
        out_type=jax.ShapeDtypeStruct((NC * n, w), jnp.float32),
        mesh=_mesh(),
        compiler_params=_SC_PARAMS,
        scratch_types=[
            pltpu.VMEM((chm, CHUNK), jnp.int32),
            pltpu.VMEM((chm, CHUNK), jnp.int32),
            pltpu.VMEM((NBUF, CHUNK, w), jnp.float32),
            pltpu.VMEM_SHARED((n + PADR, w), jnp.float32),
        ] + [pltpu.SemaphoreType.DMA] * (2 * NBUF),
    )
    def prop_k(tab_hbm, ei_hbm, out_hbm, sidx2, didx2, rows, acc, *sems):
        gsem = sems[:NBUF]
        ssem = sems[NBUF:]
        cid = lax.axis_index("c")
        sid = lax.axis_index("s")
        gid = cid * NS + sid
        ch = chb + jnp.where(gid < xtra, 1, 0)
        off = chb * gid + jnp.minimum(gid, xtra)
        pltpu.sync_copy(ei_hbm.at[0, pl.ds(off, chb), :],
                        sidx2.at[pl.ds(0, chb), :])
        pltpu.sync_copy(ei_hbm.at[1, pl.ds(off, chb), :],
                        didx2.at[pl.ds(0, chb), :])
        if xtra:
            @pl.when(gid < xtra)
            def _():
                pltpu.sync_copy(ei_hbm.at[0, pl.ds(off + chb, 1), :],
                                sidx2.at[pl.ds(chb, 1), :])
                pltpu.sync_copy(ei_hbm.at[1, pl.ds(off + chb, 1), :],
                                didx2.at[pl.ds(chb, 1), :])

        def zfill(i, carry):
            for j in range(w // 16):
                rows[0, i, pl.ds(j * 16, 16)] = jnp.zeros((16,), jnp.float32)
            return carry

        lax.fori_loop(0, CHUNK, zfill, 0)

        @pl.when(sid < nco)
        def _():
            for k in range(BN // STG2):
                pltpu.sync_copy(rows.at[0, pl.ds(0, STG2), :],
                                acc.at[pl.ds(sid * BN + k * STG2, STG2), :])

        @pl.when(sid == nco)
        def _():
            pltpu.sync_copy(rows.at[0, pl.ds(0, PADR), :],
                            acc.at[pl.ds(n, PADR), :])

        plsc.subcore_barrier()

        def gather(j, b):
            return pltpu.async_copy(tab_hbm.at[sidx2.at[j]], rows.at[b],
                                    gsem[b])

        def scatter(j, b):
            return pltpu.async_copy(rows.at[b], acc.at[didx2.at[j]],
                                    ssem[b], add=True)

        for b in range(NBUF):
            gather(b, b)

        def group(g, carry):
            base = g * NBUF
            for b in range(NBUF):
                pltpu.make_async_copy(tab_hbm.at[sidx2.at[base + b]],
                                      rows.at[b], gsem[b]).wait()
                scatter(base + b, b)
            for b in range(NBUF):
                j2 = base + NBUF + b
                pltpu.make_async_copy(rows.at[b],
                                      acc.at[pl.ds(0, CHUNK), :],
                                      ssem[b]).wait()

                @pl.when(j2 < ch)
                def _():
                    gather(j2, b)
            return carry

        nfull = ch // NBUF
        lax.fori_loop(0, nfull, group, 0)
        for b in range(NBUF):
            jt = nfull * NBUF + b

            @pl.when(jt < ch)
            def _():
                pltpu.make_async_copy(tab_hbm.at[sidx2.at[jt]],
                                      rows.at[b], gsem[b]).wait()
                scatter(jt, b)
                pltpu.make_async_copy(rows.at[b],
                                      acc.at[pl.ds(0, CHUNK), :],
                                      ssem[b]).wait()

        plsc.subcore_barrier()

        @pl.when(sid < nco)
        def _():
            for k in range(BN // STG2):
                r = sid * BN + k * STG2
                b = k % NBUF
                pltpu.sync_copy(acc.at[pl.ds(r, STG2), :],
                                rows.at[b, pl.ds(0, STG2), :])
                pltpu.sync_copy(rows.at[b, pl.ds(0, STG2), :],
                                out_hbm.at[pl.ds(cid * n + r, STG2), :])

    return prop_k


def _mm_body(x_ref, xd_ref, deg0_ref, deg1_ref, W1_ref, Wd1_ref, up_ref):
    h = W1_ref.shape[1]
    hd = Wd1_ref.shape[1]
    dinv = lax.rsqrt(1.0 + deg0_ref[...] + deg1_ref[...])
    a = jnp.dot(x_ref[...], W1_ref[...], preferred_element_type=jnp.float32)
    b = jnp.dot(xd_ref[...], Wd1_ref[...], preferred_element_type=jnp.float32)
    up_ref[:, 0:h] = a * dinv
    up_ref[:, h:h + hd] = b * dinv


def _mid_body(agg0_ref, agg1_ref, up_ref, deg0_ref, deg1_ref, b1_ref, Wc_ref,
              z_ref):
    dinv = lax.rsqrt(1.0 + deg0_ref[...] + deg1_ref[...])
    v = jnp.maximum(
        dinv * (agg0_ref[...] + agg1_ref[...] + up_ref[...]) + b1_ref[...], 0.0)
    z_ref[...] = dinv * jnp.dot(v, Wc_ref[...],
                                preferred_element_type=jnp.float32)


def _out_body(agg0_ref, agg1_ref, z_ref, deg0_ref, deg1_ref, bc_ref, o_ref):
    c = o_ref.shape[1]
    dinv = lax.rsqrt(1.0 + deg0_ref[...] + deg1_ref[...])
    pre = dinv * (agg0_ref[...] + agg1_ref[...] + z_ref[...]) + bc_ref[...]
    col = lax.broadcasted_iota(jnp.int32, pre.shape, 1)
    prem = jnp.where(col < c, pre, -1e30)
    m = jnp.max(prem, axis=1, keepdims=True)
    ex = jnp.where(col < c, jnp.exp(prem - m), 0.0)
    lse = jnp.log(jnp.sum(ex, axis=1, keepdims=True)) + m
    o_ref[...] = (pre - lse)[:, 0:c]


def _row_spec(w):
    return pl.BlockSpec((TBN, w), lambda i: (i, 0))


def _full_spec(a, b):
    return pl.BlockSpec((a, b), lambda i: (0, 0))


def kernel(x, x_deepwalk, edge_index, W1, b1, W2, b2, Wd1, bd1, Wd2, bd2):
    n, d = x.shape
    dw = x_deepwalk.shape[1]
    e = edge_index.shape[1]
    h = W1.shape[1]
    hd = Wd1.shape[1]
    c = W2.shape[1]
    wu = h + hd
    wz = 16
    f32 = jnp.float32

    if e % CHUNK:
        npad = CHUNK - e % CHUNK
        npd = jnp.arange(npad, dtype=jnp.int32)
        ei = jnp.concatenate(
            [edge_index,
             jnp.stack([npd % CHUNK, n + (npd % PADR)])], axis=1)
        e += npad
    else:
        ei = edge_index
    nrows = e // CHUNK
    ei3 = ei.reshape(2, nrows, CHUNK)

    b1cat = jnp.concatenate([b1, bd1]).reshape(1, wu)
    wc = jnp.concatenate([0.2 * W2, 0.1 * Wd2], axis=0)
    wc16 = jnp.pad(wc, ((0, 0), (0, wz - c)))
    bc16 = jnp.pad(0.2 * b2 + 0.1 * bd2, (0, wz - c)).reshape(1, wz)

    grid = n // TBN

    def _hi_spec(w):
        return pl.BlockSpec((TBN, w), lambda i: (i + grid, 0))

    deg2 = _make_deg(n, nrows)(ei3).reshape(NC * n, 1)

    up = pl.pallas_call(
        _mm_body,
        grid=(grid,),
        in_specs=[_row_spec(d), _row_spec(dw), _row_spec(1), _hi_spec(1),
                  _full_spec(d, h), _full_spec(dw, hd)],
        out_specs=_row_spec(wu),
        out_shape=jax.ShapeDtypeStruct((n, wu), f32),
    )(x, x_deepwalk, deg2, deg2, W1, Wd1)

    agg = _make_prop(n, nrows, wu)(up, ei3)

    z16 = pl.pallas_call(
        _mid_body,
        grid=(grid,),
        in_specs=[_row_spec(wu), _hi_spec(wu), _row_spec(wu),
                  _row_spec(1), _hi_spec(1),
                  _full_spec(1, wu), _full_spec(wu, wz)],
        out_specs=_row_spec(wz),
        out_shape=jax.ShapeDtypeStruct((n, wz), f32),
    )(agg, agg, up, deg2, deg2, b1cat, wc16)

    agg2 = _make_prop(n, nrows, wz)(z16, ei3)

    out = pl.pallas_call(
        _out_body,
        grid=(grid,),
        in_specs=[_row_spec(wz), _hi_spec(wz), _row_spec(wz),
                  _row_spec(1), _hi_spec(1),
                  _full_spec(1, wz)],
        out_specs=_row_spec(c),
        out_shape=jax.ShapeDtypeStruct((n, c), f32),
    )(agg2, agg2, z16, deg2, deg2, bc16)
    return out


# speedup vs baseline: 55.8339x; 1.0314x over previous
import functools

import jax
import jax.numpy as jnp
from jax import lax
from jax.experimental import pallas as pl
from jax.experimental.pallas import tpu as pltpu
from jax.experimental.pallas import tpu_sc as plsc

NC = 2
NS = 16
NW = NC * NS
CHUNK = 128
BN = 1000
TBN = 2000


def _mesh():
    return plsc.VectorSubcoreMesh(core_axis_name="c", subcore_axis_name="s")


_SC_PARAMS = pltpu.CompilerParams(use_tc_tiling_on_sc=False)


STG = 200
STG2 = 125
zb1 = 208
NBUF = 4
PADR = 128


def _make_deg(n, nrows):
    chb = nrows // NW
    xtra = nrows - chb * NW
    chm = chb + (1 if xtra else 0)
    nco = n // BN

    @functools.partial(
        pl.kernel,
        out_type=jax.ShapeDtypeStruct((NC * n,), jnp.float32),
        mesh=_mesh(),
        compiler_params=_SC_PARAMS,
        scratch_types=[
            pltpu.VMEM((chm, CHUNK), jnp.int32),
            pltpu.VMEM((CHUNK,), jnp.float32),
            pltpu.VMEM((zb1,), jnp.float32),
            pltpu.VMEM_SHARED((n + PADR,), jnp.float32),
            pltpu.SemaphoreType.DMA,
        ],
    )
    def deg_k(ei_hbm, out_hbm, didx2, ones_v, zbuf, acc, sem):
        cid = lax.axis_index("c")
        sid = lax.axis_index("s")
        gid = cid * NS + sid
        ch = chb + jnp.where(gid < xtra, 1, 0)
        off = chb * gid + jnp.minimum(gid, xtra)
        pltpu.sync_copy(ei_hbm.at[1, pl.ds(off, chb), :],
                        didx2.at[pl.ds(0, chb), :])
        if xtra:
            @pl.when(gid < xtra)
            def _():
                pltpu.sync_copy(ei_hbm.at[1, pl.ds(off + chb, 1), :],
                                didx2.at[pl.ds(chb, 1), :])

        for k in range(CHUNK // 16):
            ones_v[pl.ds(k * 16, 16)] = jnp.ones((16,), jnp.float32)
        for j in range(zb1 // 16):
            zbuf[pl.ds(j * 16, 16)] = jnp.zeros((16,), jnp.float32)

        @pl.when(sid < nco)
        def _():
            for k in range(BN // STG):
                pltpu.sync_copy(zbuf.at[pl.ds(0, STG)],
                                acc.at[pl.ds(sid * BN + k * STG, STG)])

        @pl.when(sid == nco)
        def _():
            pltpu.sync_copy(zbuf.at[pl.ds(0, PADR)], acc.at[pl.ds(n, PADR)])

        plsc.subcore_barrier()

        def fire(j, carry):
            pltpu.async_copy(ones_v, acc.at[didx2.at[j]], sem, add=True)
            return carry

        lax.fori_loop(0, ch, fire, 0)

        def drain(j, carry):
            pltpu.make_async_copy(ones_v, acc.at[pl.ds(0, CHUNK)], sem).wait()
            return carry

        lax.fori_loop(0, ch, drain, 0)
        plsc.subcore_barrier()

        @pl.when(sid < nco)
        def _():
            for k in range(BN // STG):
                r = sid * BN + k * STG
                pltpu.sync_copy(acc.at[pl.ds(r, STG)], zbuf.at[pl.ds(0, STG)])
                pltpu.sync_copy(zbuf.at[pl.ds(0, STG)],
                                out_hbm.at[pl.ds(cid * n + r, STG)])

    return deg_k


def _make_prop(n, nrows, w, nbuf=NBUF):
    chb = nrows // NW
    xtra = nrows - chb * NW
    chm = chb + (1 if xtra else 0)
    nco = n // BN

    @functools.partial(
        pl.kernel,
        out_type=jax.ShapeDtypeStruct((NC * n, w), jnp.float32),
        mesh=_mesh(),
        compiler_params=_SC_PARAMS,
        scratch_types=[
            pltpu.VMEM((chm, CHUNK), jnp.int32),
            pltpu.VMEM((chm, CHUNK), jnp.int32),
            pltpu.VMEM((nbuf, CHUNK, w), jnp.float32),
            pltpu.VMEM_SHARED((n + PADR, w), jnp.float32),
        ] + [pltpu.SemaphoreType.DMA] * (2 * nbuf),
    )
    def prop_k(tab_hbm, ei_hbm, out_hbm, sidx2, didx2, rows, acc, *sems):
        gsem = sems[:nbuf]
        ssem = sems[nbuf:]
        cid = lax.axis_index("c")
        sid = lax.axis_index("s")
        gid = cid * NS + sid
        ch = chb + jnp.where(gid < xtra, 1, 0)
        off = chb * gid + jnp.minimum(gid, xtra)
        pltpu.sync_copy(ei_hbm.at[0, pl.ds(off, chb), :],
                        sidx2.at[pl.ds(0, chb), :])
        pltpu.sync_copy(ei_hbm.at[1, pl.ds(off, chb), :],
                        didx2.at[pl.ds(0, chb), :])
        if xtra:
            @pl.when(gid < xtra)
            def _():
                pltpu.sync_copy(ei_hbm.at[0, pl.ds(off + chb, 1), :],
                                sidx2.at[pl.ds(chb, 1), :])
                pltpu.sync_copy(ei_hbm.at[1, pl.ds(off + chb, 1), :],
                                didx2.at[pl.ds(chb, 1), :])

        def zfill(i, carry):
            for j in range(w // 16):
                rows[0, i, pl.ds(j * 16, 16)] = jnp.zeros((16,), jnp.float32)
            return carry

        lax.fori_loop(0, CHUNK, zfill, 0)

        @pl.when(sid < nco)
        def _():
            for k in range(BN // STG2):
                pltpu.sync_copy(rows.at[0, pl.ds(0, STG2), :],
                                acc.at[pl.ds(sid * BN + k * STG2, STG2), :])

        @pl.when(sid == nco)
        def _():
            pltpu.sync_copy(rows.at[0, pl.ds(0, PADR), :],
                            acc.at[pl.ds(n, PADR), :])

        plsc.subcore_barrier()

        def gather(j, b):
            return pltpu.async_copy(tab_hbm.at[sidx2.at[j]], rows.at[b],
                                    gsem[b])

        def scatter(j, b):
            return pltpu.async_copy(rows.at[b], acc.at[didx2.at[j]],
                                    ssem[b], add=True)

        for b in range(nbuf):
            gather(b, b)

        def group(g, carry):
            base = g * nbuf
            for b in range(nbuf):
                pltpu.make_async_copy(tab_hbm.at[sidx2.at[base + b]],
                                      rows.at[b], gsem[b]).wait()
                scatter(base + b, b)
            for b in range(nbuf):
                j2 = base + nbuf + b
                pltpu.make_async_copy(rows.at[b],
                                      acc.at[pl.ds(0, CHUNK), :],
                                      ssem[b]).wait()

                @pl.when(j2 < ch)
                def _():
                    gather(j2, b)
            return carry

        nfull = ch // nbuf
        lax.fori_loop(0, nfull, group, 0)
        for b in range(nbuf):
            jt = nfull * nbuf + b

            @pl.when(jt < ch)
            def _():
                pltpu.make_async_copy(tab_hbm.at[sidx2.at[jt]],
                                      rows.at[b], gsem[b]).wait()
                scatter(jt, b)
                pltpu.make_async_copy(rows.at[b],
                                      acc.at[pl.ds(0, CHUNK), :],
                                      ssem[b]).wait()

        plsc.subcore_barrier()

        @pl.when(sid < nco)
        def _():
            for k in range(BN // STG2):
                r = sid * BN + k * STG2
                b = k % nbuf
                pltpu.sync_copy(acc.at[pl.ds(r, STG2), :],
                                rows.at[b, pl.ds(0, STG2), :])
                pltpu.sync_copy(rows.at[b, pl.ds(0, STG2), :],
                                out_hbm.at[pl.ds(cid * n + r, STG2), :])

    return prop_k


def _mm_body(x_ref, xd_ref, deg0_ref, deg1_ref, W1_ref, Wd1_ref, up_ref):
    h = W1_ref.shape[1]
    hd = Wd1_ref.shape[1]
    dinv = lax.rsqrt(1.0 + deg0_ref[...] + deg1_ref[...])
    a = jnp.dot(x_ref[...], W1_ref[...], preferred_element_type=jnp.float32)
    b = jnp.dot(xd_ref[...], Wd1_ref[...], preferred_element_type=jnp.float32)
    up_ref[:, 0:h] = a * dinv
    up_ref[:, h:h + hd] = b * dinv


def _mid_body(agg0_ref, agg1_ref, up_ref, deg0_ref, deg1_ref, b1_ref, Wc_ref,
              z_ref):
    dinv = lax.rsqrt(1.0 + deg0_ref[...] + deg1_ref[...])
    v = jnp.maximum(
        dinv * (agg0_ref[...] + agg1_ref[...] + up_ref[...]) + b1_ref[...], 0.0)
    z_ref[...] = dinv * jnp.dot(v, Wc_ref[...],
                                preferred_element_type=jnp.float32)


def _out_body(agg0_ref, agg1_ref, z_ref, deg0_ref, deg1_ref, bc_ref, o_ref):
    c = o_ref.shape[1]
    dinv = lax.rsqrt(1.0 + deg0_ref[...] + deg1_ref[...])
    pre = dinv * (agg0_ref[...] + agg1_ref[...] + z_ref[...]) + bc_ref[...]
    col = lax.broadcasted_iota(jnp.int32, pre.shape, 1)
    prem = jnp.where(col < c, pre, -1e30)
    m = jnp.max(prem, axis=1, keepdims=True)
    ex = jnp.where(col < c, jnp.exp(prem - m), 0.0)
    lse = jnp.log(jnp.sum(ex, axis=1, keepdims=True)) + m
    o_ref[...] = (pre - lse)[:, 0:c]


def _row_spec(w):
    return pl.BlockSpec((TBN, w), lambda i: (i, 0))


def _full_spec(a, b):
    return pl.BlockSpec((a, b), lambda i: (0, 0))


def kernel(x, x_deepwalk, edge_index, W1, b1, W2, b2, Wd1, bd1, Wd2, bd2):
    n, d = x.shape
    dw = x_deepwalk.shape[1]
    e = edge_index.shape[1]
    h = W1.shape[1]
    hd = Wd1.shape[1]
    c = W2.shape[1]
    wu = h + hd
    wz = 16
    f32 = jnp.float32

    if e % CHUNK:
        npad = CHUNK - e % CHUNK
        npd = jnp.arange(npad, dtype=jnp.int32)
        ei = jnp.concatenate(
            [edge_index,
             jnp.stack([npd % CHUNK, n + (npd % PADR)])], axis=1)
        e += npad
    else:
        ei = edge_index
    nrows = e // CHUNK
    ei3 = ei.reshape(2, nrows, CHUNK)

    b1cat = jnp.concatenate([b1, bd1]).reshape(1, wu)
    wc = jnp.concatenate([0.2 * W2, 0.1 * Wd2], axis=0)
    wc16 = jnp.pad(wc, ((0, 0), (0, wz - c)))
    bc16 = jnp.pad(0.2 * b2 + 0.1 * bd2, (0, wz - c)).reshape(1, wz)

    grid = n // TBN

    def _hi_spec(w):
        return pl.BlockSpec((TBN, w), lambda i: (i + grid, 0))

    deg2 = _make_deg(n, nrows)(ei3).reshape(NC * n, 1)

    up = pl.pallas_call(
        _mm_body,
        grid=(grid,),
        in_specs=[_row_spec(d), _row_spec(dw), _row_spec(1), _hi_spec(1),
                  _full_spec(d, h), _full_spec(dw, hd)],
        out_specs=_row_spec(wu),
        out_shape=jax.ShapeDtypeStruct((n, wu), f32),
    )(x, x_deepwalk, deg2, deg2, W1, Wd1)

    agg = _make_prop(n, nrows, wu)(up, ei3)

    z16 = pl.pallas_call(
        _mid_body,
        grid=(grid,),
        in_specs=[_row_spec(wu), _hi_spec(wu), _row_spec(wu),
                  _row_spec(1), _hi_spec(1),
                  _full_spec(1, wu), _full_spec(wu, wz)],
        out_specs=_row_spec(wz),
        out_shape=jax.ShapeDtypeStruct((n, wz), f32),
    )(agg, agg, up, deg2, deg2, b1cat, wc16)

    agg2 = _make_prop(n, nrows, wz, nbuf=8)(z16, ei3)

    out = pl.pallas_call(
        _out_body,
        grid=(grid,),
        in_specs=[_row_spec(wz), _hi_spec(wz), _row_spec(wz),
                  _row_spec(1), _hi_spec(1),
                  _full_spec(1, wz)],
        out_specs=_row_spec(c),
        out_shape=jax.ShapeDtypeStruct((n, c), f32),
    )(agg2, agg2, z16, deg2, deg2, bc16)
    return out
